# Initial kernel scaffold; baseline (speedup 1.0000x reference)
import jax
import jax.numpy as jnp
from jax.experimental import pallas as pl


def kernel(adj, features, edge_type, weight, q, k, bias):
    raise NotImplementedError("write your pallas kernel here")


# speedup vs baseline: 16.6685x; 16.6685x over previous
import jax
import jax.numpy as jnp
from jax import lax
from jax.experimental import pallas as pl
from jax.experimental.pallas import tpu as pltpu
from jax.experimental.pallas import tpu_sc as plsc

N = 10000
E = 320000
DIM = 128
R = 8

NC = 2
NS = 16
NW = NC * NS
NPAD = 10240
CB = 128
CHUNKS = 79
EW = CB * CHUNKS
EP = EW * NW
BN = 512
NT = NPAD // BN
RPS = NPAD // NS


def _xw_body(feat_ref, w_ref, qk_ref, xw_ref, qnk_ref):
    xb = jnp.dot(feat_ref[...], w_ref[0], preferred_element_type=jnp.float32)
    xw_ref[0] = xb
    qnk_ref[0] = jnp.dot(xb, qk_ref[...], preferred_element_type=jnp.float32)


_xw_call = pl.pallas_call(
    _xw_body,
    grid=(NT, R),
    in_specs=[
        pl.BlockSpec((BN, DIM), lambda t, r: (t, 0)),
        pl.BlockSpec((1, DIM, DIM), lambda t, r: (r, 0, 0)),
        pl.BlockSpec((DIM, 2), lambda t, r: (0, 0)),
    ],
    out_specs=[
        pl.BlockSpec((1, BN, DIM), lambda t, r: (r, t, 0)),
        pl.BlockSpec((1, BN, 2), lambda t, r: (r, t, 0)),
    ],
    out_shape=[
        jax.ShapeDtypeStruct((R, NPAD, DIM), jnp.float32),
        jax.ShapeDtypeStruct((R, NPAD, 2), jnp.float32),
    ],
)


def _sc_body(xwf, qflat, kflat, idxj_h, idxq_h, dst_h, part, den_out,
             idxj_v, idxq_v, dst_v, qi_v, kj_v, rows, srows, den_v,
             accum, sem):
    cid = lax.axis_index("c")
    sid = lax.axis_index("s")
    w = sid * NC + cid
    lane = lax.iota(jnp.int32, 16)
    zero16 = jnp.zeros((16,), jnp.float32)

    def _zden(i, c):
        den_v[pl.ds(i * 16, 16)] = zero16
        return c

    lax.fori_loop(0, NPAD // 16, _zden, 0)

    def _zrow(i, c):
        for h in range(DIM // 16):
            srows[i, h * 16:(h + 1) * 16] = zero16
        return c

    lax.fori_loop(0, CB, _zrow, 0)
    for b in range(RPS // CB):
        pltpu.sync_copy(srows, accum.at[pl.ds(sid * RPS + b * CB, CB)])
    plsc.subcore_barrier()

    def _chunk(t, c):
        base = w * EW + t * CB
        pltpu.sync_copy(idxj_h.at[pl.ds(base, CB)], idxj_v)
        pltpu.sync_copy(idxq_h.at[pl.ds(base, CB)], idxq_v)
        pltpu.sync_copy(dst_h.at[pl.ds(base, CB)], dst_v)
        cp1 = pltpu.async_copy(qflat.at[idxq_v], qi_v, sem)
        cp2 = pltpu.async_copy(kflat.at[idxj_v], kj_v, sem)
        cp3 = pltpu.async_copy(xwf.at[idxj_v], rows, sem)
        cp1.wait()
        cp2.wait()
        cp3.wait()
        for g in range(CB // 16):
            qi = qi_v[g * 16:(g + 1) * 16]
            kj = kj_v[g * 16:(g + 1) * 16]
            s = qi + kj
            a = jnp.where(s >= 0.0, s, 0.2 * s)
            ex = jnp.exp(a)
            d16 = dst_v[g * 16:(g + 1) * 16]
            for l in range(16):
                i = g * 16 + l
                sv = jnp.full((16,), ex[l], jnp.float32)
                for h in range(DIM // 16):
                    srows[i, h * 16:(h + 1) * 16] = (
                        rows[i, h * 16:(h + 1) * 16] * sv)
                d = d16[l]
                wb = (d >> 4) << 4
                v = den_v[pl.ds(wb, 16)]
                den_v[pl.ds(wb, 16)] = v + jnp.where(lane == d - wb, sv, 0.0)
        pltpu.sync_copy(srows, accum.at[dst_v], add=True)
        return c

    lax.fori_loop(0, CHUNKS, _chunk, 0)

    pltpu.sync_copy(den_v, den_out.at[w])
    plsc.subcore_barrier()

    for b in range(RPS // CB):
        off = sid * RPS + b * CB
        pltpu.sync_copy(accum.at[pl.ds(off, CB)], srows)
        pltpu.sync_copy(srows, part.at[cid, pl.ds(off, CB)])


_sc_call = pl.kernel(
    _sc_body,
    out_type=[
        jax.ShapeDtypeStruct((NC, NPAD, DIM), jnp.float32),
        jax.ShapeDtypeStruct((NW, NPAD), jnp.float32),
    ],
    mesh=plsc.VectorSubcoreMesh(core_axis_name="c", subcore_axis_name="s"),
    scratch_types=[
        pltpu.VMEM((CB,), jnp.int32),
        pltpu.VMEM((CB,), jnp.int32),
        pltpu.VMEM((CB,), jnp.int32),
        pltpu.VMEM((CB,), jnp.float32),
        pltpu.VMEM((CB,), jnp.float32),
        pltpu.VMEM((CB, DIM), jnp.float32),
        pltpu.VMEM((CB, DIM), jnp.float32),
        pltpu.VMEM((NPAD,), jnp.float32),
        pltpu.VMEM_SHARED((NPAD, DIM), jnp.float32),
        pltpu.SemaphoreType.DMA,
    ],
)


def _fin_body(part_ref, den_ref, bias_ref, out_ref):
    num = part_ref[0] + part_ref[1]
    den = jnp.sum(den_ref[...], axis=1, keepdims=True)
    out_ref[...] = jnp.maximum(num / (den + 1e-16) + bias_ref[...], 0.0)


_fin_call = pl.pallas_call(
    _fin_body,
    grid=(NT,),
    in_specs=[
        pl.BlockSpec((NC, BN, DIM), lambda t: (0, t, 0)),
        pl.BlockSpec((BN, NW), lambda t: (t, 0)),
        pl.BlockSpec((1, DIM), lambda t: (0, 0)),
    ],
    out_specs=pl.BlockSpec((BN, DIM), lambda t: (t, 0)),
    out_shape=jax.ShapeDtypeStruct((NPAD, DIM), jnp.float32),
)


def kernel(adj, features, edge_type, weight, q, k, bias):
    src = adj[0]
    dst = adj[1]
    featp = jnp.zeros((NPAD, DIM), jnp.float32).at[:N].set(features)
    qk2 = jnp.concatenate([q, k], axis=1)
    xw, qnk = _xw_call(featp, weight, qk2)
    xwf = xw.reshape(R * NPAD, DIM)
    qflat = qnk[:, :, 0].reshape(R * NPAD)
    kflat = qnk[:, :, 1].reshape(R * NPAD)
    pad = EP - E
    srcp = jnp.concatenate([src, jnp.zeros((pad,), jnp.int32)])
    dstp = jnp.concatenate([dst, jnp.full((pad,), NPAD - 1, jnp.int32)])
    relp = jnp.concatenate([edge_type, jnp.zeros((pad,), jnp.int32)])
    idxj = relp * NPAD + srcp
    idxq = relp * NPAD + dstp
    part, den_all = _sc_call(xwf, qflat, kflat, idxj, idxq, dstp)
    out = _fin_call(part, den_all.T, bias.reshape(1, DIM))
    return out[:N]


# speedup vs baseline: 16.9802x; 1.0187x over previous
import jax
import jax.numpy as jnp
from jax import lax
from jax.experimental import pallas as pl
from jax.experimental.pallas import tpu as pltpu
from jax.experimental.pallas import tpu_sc as plsc

N = 10000
E = 320000
DIM = 128
R = 8

NC = 2
NS = 16
NW = NC * NS
NPAD = 10240
CB = 80
CHUNKS = 128
EW = CB * CHUNKS
EP = EW * NW
BN = 512
NT = NPAD // BN
RPS = NPAD // NS


def _xw_body(feat_ref, w_ref, qk_ref, xw_ref, qnk_ref):
    xb = jnp.dot(feat_ref[...], w_ref[0], preferred_element_type=jnp.float32)
    xw_ref[0] = xb
    qnk_ref[0] = jnp.dot(xb, qk_ref[...], preferred_element_type=jnp.float32)


_xw_call = pl.pallas_call(
    _xw_body,
    grid=(NT, R),
    in_specs=[
        pl.BlockSpec((BN, DIM), lambda t, r: (t, 0)),
        pl.BlockSpec((1, DIM, DIM), lambda t, r: (r, 0, 0)),
        pl.BlockSpec((DIM, 2), lambda t, r: (0, 0)),
    ],
    out_specs=[
        pl.BlockSpec((1, BN, DIM), lambda t, r: (r, t, 0)),
        pl.BlockSpec((1, BN, 2), lambda t, r: (r, t, 0)),
    ],
    out_shape=[
        jax.ShapeDtypeStruct((R, NPAD, DIM), jnp.float32),
        jax.ShapeDtypeStruct((R, NPAD, 2), jnp.float32),
    ],
)


def _sc_body(xwf, qflat, kflat, idxj_h, idxq_h, dst_h, part, den_out,
             idxj_a, idxj_b, idxq_a, idxq_b, dst_a, dst_b,
             qi_a, qi_b, kj_a, kj_b, rows_a, rows_b,
             srows, den_v, accum, sem_a, sem_b):
    cid = lax.axis_index("c")
    sid = lax.axis_index("s")
    w = sid * NC + cid
    lane = lax.iota(jnp.int32, 16)
    zero16 = jnp.zeros((16,), jnp.float32)

    def _zden(i, c):
        den_v[pl.ds(i * 16, 16)] = zero16
        return c

    lax.fori_loop(0, NPAD // 16, _zden, 0)

    def _zrow(i, c):
        for h in range(DIM // 16):
            srows[i, h * 16:(h + 1) * 16] = zero16
        return c

    lax.fori_loop(0, CB, _zrow, 0)
    for b in range(RPS // CB):
        pltpu.sync_copy(srows, accum.at[pl.ds(sid * RPS + b * CB, CB)])
    plsc.subcore_barrier()

    def _lin(t, idxj_v, idxq_v, dst_v):
        base = w * EW + t * CB
        pltpu.sync_copy(idxj_h.at[pl.ds(base, CB)], idxj_v)
        pltpu.sync_copy(idxq_h.at[pl.ds(base, CB)], idxq_v)
        pltpu.sync_copy(dst_h.at[pl.ds(base, CB)], dst_v)

    def _fire(idxj_v, idxq_v, qi_v, kj_v, rows, sem):
        pltpu.async_copy(qflat.at[idxq_v], qi_v, sem)
        pltpu.async_copy(kflat.at[idxj_v], kj_v, sem)
        pltpu.async_copy(xwf.at[idxj_v], rows, sem)

    def _drain(qi_v, kj_v, rows, sem):
        pltpu.make_async_copy(qflat.at[idxq_a], qi_v, sem).wait()
        pltpu.make_async_copy(kflat.at[idxj_a], kj_v, sem).wait()
        pltpu.make_async_copy(xwf.at[idxj_a], rows, sem).wait()

    def _compute(qi_v, kj_v, rows, dst_v):
        for g in range(CB // 16):
            qi = qi_v[g * 16:(g + 1) * 16]
            kj = kj_v[g * 16:(g + 1) * 16]
            s = qi + kj
            a = jnp.where(s >= 0.0, s, 0.2 * s)
            ex = jnp.exp(a)
            d16 = dst_v[g * 16:(g + 1) * 16]
            for l in range(16):
                i = g * 16 + l
                sv = jnp.full((16,), ex[l], jnp.float32)
                for h in range(DIM // 16):
                    srows[i, h * 16:(h + 1) * 16] = (
                        rows[i, h * 16:(h + 1) * 16] * sv)
                d = d16[l]
                wb = (d >> 4) << 4
                v = den_v[pl.ds(wb, 16)]
                den_v[pl.ds(wb, 16)] = v + jnp.where(lane == d - wb, sv, 0.0)
        pltpu.sync_copy(srows, accum.at[dst_v], add=True)

    _lin(0, idxj_a, idxq_a, dst_a)
    _fire(idxj_a, idxq_a, qi_a, kj_a, rows_a, sem_a)

    def _pair(u, c):
        t0 = 2 * u
        _lin(t0 + 1, idxj_b, idxq_b, dst_b)
        _fire(idxj_b, idxq_b, qi_b, kj_b, rows_b, sem_b)
        _drain(qi_a, kj_a, rows_a, sem_a)
        _compute(qi_a, kj_a, rows_a, dst_a)
        _lin(t0 + 2, idxj_a, idxq_a, dst_a)
        _fire(idxj_a, idxq_a, qi_a, kj_a, rows_a, sem_a)
        _drain(qi_b, kj_b, rows_b, sem_b)
        _compute(qi_b, kj_b, rows_b, dst_b)
        return c

    lax.fori_loop(0, CHUNKS // 2, _pair, 0)
    _drain(qi_a, kj_a, rows_a, sem_a)

    pltpu.sync_copy(den_v, den_out.at[w])
    plsc.subcore_barrier()

    for b in range(RPS // CB):
        off = sid * RPS + b * CB
        pltpu.sync_copy(accum.at[pl.ds(off, CB)], srows)
        pltpu.sync_copy(srows, part.at[cid, pl.ds(off, CB)])


_sc_call = pl.kernel(
    _sc_body,
    out_type=[
        jax.ShapeDtypeStruct((NC, NPAD, DIM), jnp.float32),
        jax.ShapeDtypeStruct((NW, NPAD), jnp.float32),
    ],
    mesh=plsc.VectorSubcoreMesh(core_axis_name="c", subcore_axis_name="s"),
    scratch_types=[
        pltpu.VMEM((CB,), jnp.int32),
        pltpu.VMEM((CB,), jnp.int32),
        pltpu.VMEM((CB,), jnp.int32),
        pltpu.VMEM((CB,), jnp.int32),
        pltpu.VMEM((CB,), jnp.int32),
        pltpu.VMEM((CB,), jnp.int32),
        pltpu.VMEM((CB,), jnp.float32),
        pltpu.VMEM((CB,), jnp.float32),
        pltpu.VMEM((CB,), jnp.float32),
        pltpu.VMEM((CB,), jnp.float32),
        pltpu.VMEM((CB, DIM), jnp.float32),
        pltpu.VMEM((CB, DIM), jnp.float32),
        pltpu.VMEM((CB, DIM), jnp.float32),
        pltpu.VMEM((NPAD,), jnp.float32),
        pltpu.VMEM_SHARED((NPAD, DIM), jnp.float32),
        pltpu.SemaphoreType.DMA,
        pltpu.SemaphoreType.DMA,
    ],
)


def _fin_body(part_ref, den_ref, bias_ref, out_ref):
    num = part_ref[0] + part_ref[1]
    den = jnp.sum(den_ref[...], axis=1, keepdims=True)
    out_ref[...] = jnp.maximum(num / (den + 1e-16) + bias_ref[...], 0.0)


_fin_call = pl.pallas_call(
    _fin_body,
    grid=(NT,),
    in_specs=[
        pl.BlockSpec((NC, BN, DIM), lambda t: (0, t, 0)),
        pl.BlockSpec((BN, NW), lambda t: (t, 0)),
        pl.BlockSpec((1, DIM), lambda t: (0, 0)),
    ],
    out_specs=pl.BlockSpec((BN, DIM), lambda t: (t, 0)),
    out_shape=jax.ShapeDtypeStruct((NPAD, DIM), jnp.float32),
)


def kernel(adj, features, edge_type, weight, q, k, bias):
    src = adj[0]
    dst = adj[1]
    featp = jnp.zeros((NPAD, DIM), jnp.float32).at[:N].set(features)
    qk2 = jnp.concatenate([q, k], axis=1)
    xw, qnk = _xw_call(featp, weight, qk2)
    xwf = xw.reshape(R * NPAD, DIM)
    qflat = qnk[:, :, 0].reshape(R * NPAD)
    kflat = qnk[:, :, 1].reshape(R * NPAD)
    pad = EP + CB - E
    srcp = jnp.concatenate([src, jnp.zeros((pad,), jnp.int32)])
    dstp = jnp.concatenate([dst, jnp.full((pad,), NPAD - 1, jnp.int32)])
    relp = jnp.concatenate([edge_type, jnp.zeros((pad,), jnp.int32)])
    idxj = relp * NPAD + srcp
    idxq = relp * NPAD + dstp
    part, den_all = _sc_call(xwf, qflat, kflat, idxj, idxq, dstp)
    out = _fin_call(part, den_all.T, bias.reshape(1, DIM))
    return out[:N]


# speedup vs baseline: 19.3101x; 1.1372x over previous
import jax
import jax.numpy as jnp
from jax import lax
from jax.experimental import pallas as pl
from jax.experimental.pallas import tpu as pltpu
from jax.experimental.pallas import tpu_sc as plsc

N = 10000
E = 320000
DIM = 128
R = 8

NC = 2
NS = 16
NW = NC * NS
NPAD = 10240
CB = 80
CHUNKS = 128
EW = CB * CHUNKS
EP = EW * NW
BN = 512
NT = NPAD // BN
RPS = NPAD // NS


def _xw_body(feat_ref, w_ref, qk_ref, xw_ref, qnk_ref):
    xb = jnp.dot(feat_ref[...], w_ref[0], preferred_element_type=jnp.float32)
    xw_ref[0] = xb
    qnk_ref[0] = jnp.dot(xb, qk_ref[...], preferred_element_type=jnp.float32)


_xw_call = pl.pallas_call(
    _xw_body,
    grid=(NT, R),
    in_specs=[
        pl.BlockSpec((BN, DIM), lambda t, r: (t, 0)),
        pl.BlockSpec((1, DIM, DIM), lambda t, r: (r, 0, 0)),
        pl.BlockSpec((DIM, 2), lambda t, r: (0, 0)),
    ],
    out_specs=[
        pl.BlockSpec((1, BN, DIM), lambda t, r: (r, t, 0)),
        pl.BlockSpec((1, BN, 2), lambda t, r: (r, t, 0)),
    ],
    out_shape=[
        jax.ShapeDtypeStruct((R, NPAD, DIM), jnp.float32),
        jax.ShapeDtypeStruct((R, NPAD, 2), jnp.float32),
    ],
)


def _sc_body(xwf, qflat, kflat, idxj_h, idxq_h, dst_h, part, den_out,
             idxj_a, idxq_a, dst_a, idxj_b, idxq_b, dst_b,
             idxj_c, idxq_c, dst_c, idxj_d, idxq_d, dst_d,
             qi_a, qi_b, kj_a, kj_b, rows_a, rows_b, srows, den_v, accum,
             sem_l0, sem_l1, sem_l2, sem_l3, sem_a, sem_b):
    cid = lax.axis_index("c")
    sid = lax.axis_index("s")
    w = sid * NC + cid
    lane = lax.iota(jnp.int32, 16)
    zero16 = jnp.zeros((16,), jnp.float32)

    def _zden(i, c):
        den_v[pl.ds(i * 16, 16)] = zero16
        return c

    lax.fori_loop(0, NPAD // 16, _zden, 0)

    def _zrow(i, c):
        for h in range(DIM // 16):
            srows[i, h * 16:(h + 1) * 16] = zero16
        return c

    lax.fori_loop(0, CB, _zrow, 0)
    for b in range(RPS // CB):
        pltpu.sync_copy(srows, accum.at[pl.ds(sid * RPS + b * CB, CB)])
    plsc.subcore_barrier()

    def _lin(t, slot):
        idxj_v, idxq_v, dst_v, sem = slot
        base = w * EW + t * CB
        pltpu.async_copy(idxj_h.at[pl.ds(base, CB)], idxj_v, sem)
        pltpu.async_copy(idxq_h.at[pl.ds(base, CB)], idxq_v, sem)
        pltpu.async_copy(dst_h.at[pl.ds(base, CB)], dst_v, sem)

    def _lin_drain(slot):
        idxj_v, idxq_v, dst_v, sem = slot
        pltpu.make_async_copy(idxj_h.at[pl.ds(0, CB)], idxj_v, sem).wait()
        pltpu.make_async_copy(idxj_h.at[pl.ds(0, CB)], idxq_v, sem).wait()
        pltpu.make_async_copy(idxj_h.at[pl.ds(0, CB)], dst_v, sem).wait()

    def _fire(slot, rbuf):
        idxj_v, idxq_v, _, _ = slot
        qi_v, kj_v, rows, sem = rbuf
        pltpu.async_copy(qflat.at[idxq_v], qi_v, sem)
        pltpu.async_copy(kflat.at[idxj_v], kj_v, sem)
        pltpu.async_copy(xwf.at[idxj_v], rows, sem)

    def _drain(rbuf):
        qi_v, kj_v, rows, sem = rbuf
        pltpu.make_async_copy(qflat.at[pl.ds(0, CB)], qi_v, sem).wait()
        pltpu.make_async_copy(qflat.at[pl.ds(0, CB)], kj_v, sem).wait()
        pltpu.make_async_copy(xwf.at[pl.ds(0, CB)], rows, sem).wait()

    def _compute(rbuf, slot):
        qi_v, kj_v, rows, _ = rbuf
        dst_v = slot[2]
        for g in range(CB // 16):
            qi = qi_v[g * 16:(g + 1) * 16]
            kj = kj_v[g * 16:(g + 1) * 16]
            s = qi + kj
            a = jnp.where(s >= 0.0, s, 0.2 * s)
            ex = jnp.exp(a)
            d16 = dst_v[g * 16:(g + 1) * 16]
            for l in range(16):
                i = g * 16 + l
                sv = jnp.full((16,), ex[l], jnp.float32)
                for h in range(DIM // 16):
                    srows[i, h * 16:(h + 1) * 16] = (
                        rows[i, h * 16:(h + 1) * 16] * sv)
                d = d16[l]
                wb = (d >> 4) << 4
                v = den_v[pl.ds(wb, 16)]
                den_v[pl.ds(wb, 16)] = v + jnp.where(lane == d - wb, sv, 0.0)
        pltpu.sync_copy(srows, accum.at[dst_v], add=True)

    slots = [(idxj_a, idxq_a, dst_a, sem_l0), (idxj_b, idxq_b, dst_b, sem_l1),
             (idxj_c, idxq_c, dst_c, sem_l2), (idxj_d, idxq_d, dst_d, sem_l3)]
    rbufs = [(qi_a, kj_a, rows_a, sem_a), (qi_b, kj_b, rows_b, sem_b)]
    _lin(0, slots[0])
    _lin(1, slots[1])
    _lin_drain(slots[0])
    _fire(slots[0], rbufs[0])

    def _quad(u, c):
        for kk in range(4):
            t = 4 * u + kk
            _lin(t + 2, slots[(kk + 2) % 4])
            _lin_drain(slots[(kk + 1) % 4])
            _fire(slots[(kk + 1) % 4], rbufs[(kk + 1) % 2])
            _drain(rbufs[kk % 2])
            _compute(rbufs[kk % 2], slots[kk % 4])
        return c

    lax.fori_loop(0, CHUNKS // 4, _quad, 0)
    _drain(rbufs[0])
    _lin_drain(slots[1])

    pltpu.sync_copy(den_v, den_out.at[w])
    plsc.subcore_barrier()

    for b in range(RPS // CB):
        off = sid * RPS + b * CB
        pltpu.sync_copy(accum.at[pl.ds(off, CB)], srows)
        pltpu.sync_copy(srows, part.at[cid, pl.ds(off, CB)])


_sc_call = pl.kernel(
    _sc_body,
    out_type=[
        jax.ShapeDtypeStruct((NC, NPAD, DIM), jnp.float32),
        jax.ShapeDtypeStruct((NW, NPAD), jnp.float32),
    ],
    mesh=plsc.VectorSubcoreMesh(core_axis_name="c", subcore_axis_name="s"),
    scratch_types=(
        [pltpu.VMEM((CB,), jnp.int32)] * 12
        + [pltpu.VMEM((CB,), jnp.float32)] * 4
        + [pltpu.VMEM((CB, DIM), jnp.float32)] * 3
        + [pltpu.VMEM((NPAD,), jnp.float32),
           pltpu.VMEM_SHARED((NPAD, DIM), jnp.float32)]
        + [pltpu.SemaphoreType.DMA] * 6
    ),
)


def _fin_body(part_ref, den_ref, bias_ref, out_ref):
    num = part_ref[0] + part_ref[1]
    den = jnp.sum(den_ref[...], axis=1, keepdims=True)
    out_ref[...] = jnp.maximum(num / (den + 1e-16) + bias_ref[...], 0.0)


_fin_call = pl.pallas_call(
    _fin_body,
    grid=(NT,),
    in_specs=[
        pl.BlockSpec((NC, BN, DIM), lambda t: (0, t, 0)),
        pl.BlockSpec((BN, NW), lambda t: (t, 0)),
        pl.BlockSpec((1, DIM), lambda t: (0, 0)),
    ],
    out_specs=pl.BlockSpec((BN, DIM), lambda t: (t, 0)),
    out_shape=jax.ShapeDtypeStruct((NPAD, DIM), jnp.float32),
)


def kernel(adj, features, edge_type, weight, q, k, bias):
    src = adj[0]
    dst = adj[1]
    featp = jnp.zeros((NPAD, DIM), jnp.float32).at[:N].set(features)
    qk2 = jnp.concatenate([q, k], axis=1)
    xw, qnk = _xw_call(featp, weight, qk2)
    xwf = xw.reshape(R * NPAD, DIM)
    qflat = qnk[:, :, 0].reshape(R * NPAD)
    kflat = qnk[:, :, 1].reshape(R * NPAD)
    pad = EP + 2 * CB - E
    srcp = jnp.concatenate([src, jnp.zeros((pad,), jnp.int32)])
    dstp = jnp.concatenate([dst, jnp.full((pad,), NPAD - 1, jnp.int32)])
    relp = jnp.concatenate([edge_type, jnp.zeros((pad,), jnp.int32)])
    idxj = relp * NPAD + srcp
    idxq = relp * NPAD + dstp
    part, den_all = _sc_call(xwf, qflat, kflat, idxj, idxq, dstp)
    out = _fin_call(part, den_all.T, bias.reshape(1, DIM))
    return out[:N]


# speedup vs baseline: 21.3668x; 1.1065x over previous
import jax
import jax.numpy as jnp
from jax import lax
from jax.experimental import pallas as pl
from jax.experimental.pallas import tpu as pltpu
from jax.experimental.pallas import tpu_sc as plsc

N = 10000
E = 320000
DIM = 128
R = 8

NC = 2
NS = 16
NW = NC * NS
NPAD = 10240
CB = 64
CHUNKS = 160
EW = CB * CHUNKS
EP = EW * NW
BN = 512
NT = NPAD // BN
RPS = NPAD // NS


def _xw_body(feat_ref, w_ref, qk_ref, xw_ref, qnk_ref):
    xb = jnp.dot(feat_ref[...], w_ref[0], preferred_element_type=jnp.float32)
    xw_ref[0] = xb
    qnk_ref[0] = jnp.dot(xb, qk_ref[...], preferred_element_type=jnp.float32)


_xw_call = pl.pallas_call(
    _xw_body,
    grid=(NT, R),
    in_specs=[
        pl.BlockSpec((BN, DIM), lambda t, r: (t, 0)),
        pl.BlockSpec((1, DIM, DIM), lambda t, r: (r, 0, 0)),
        pl.BlockSpec((DIM, 2), lambda t, r: (0, 0)),
    ],
    out_specs=[
        pl.BlockSpec((1, BN, DIM), lambda t, r: (r, t, 0)),
        pl.BlockSpec((1, BN, 2), lambda t, r: (r, t, 0)),
    ],
    out_shape=[
        jax.ShapeDtypeStruct((R, NPAD, DIM), jnp.float32),
        jax.ShapeDtypeStruct((R, NPAD, 2), jnp.float32),
    ],
)


def _sc_body(xwf, qflat, kflat, idxj_h, idxq_h, dst_h, part, den_out,
             idxj_a, idxq_a, dst_a, idxj_b, idxq_b, dst_b,
             idxj_c, idxq_c, dst_c, idxj_d, idxq_d, dst_d, prime_idx,
             qi_a, qi_b, kj_a, kj_b, rows_a, rows_b, srows_a, srows_b,
             den_v, accum,
             sem_l0, sem_l1, sem_l2, sem_l3, sem_a, sem_b, sem_s0, sem_s1):
    cid = lax.axis_index("c")
    sid = lax.axis_index("s")
    w = sid * NC + cid
    lane = lax.iota(jnp.int32, 16)
    zero16 = jnp.zeros((16,), jnp.float32)

    def _zden(i, c):
        den_v[pl.ds(i * 16, 16)] = zero16
        return c

    lax.fori_loop(0, NPAD // 16, _zden, 0)

    def _zrow(i, c):
        for h in range(DIM // 16):
            srows_a[i, h * 16:(h + 1) * 16] = zero16
            srows_b[i, h * 16:(h + 1) * 16] = zero16
        return c

    lax.fori_loop(0, CB, _zrow, 0)
    for h in range(CB // 16):
        prime_idx[h * 16:(h + 1) * 16] = jnp.zeros((16,), jnp.int32)
    for b in range(RPS // CB):
        pltpu.sync_copy(srows_a, accum.at[pl.ds(sid * RPS + b * CB, CB)])
    plsc.subcore_barrier()

    def _lin(t, slot):
        idxj_v, idxq_v, dst_v, sem = slot
        base = w * EW + t * CB
        pltpu.async_copy(idxj_h.at[pl.ds(base, CB)], idxj_v, sem)
        pltpu.async_copy(idxq_h.at[pl.ds(base, CB)], idxq_v, sem)
        pltpu.async_copy(dst_h.at[pl.ds(base, CB)], dst_v, sem)

    def _lin_drain(slot):
        idxj_v, idxq_v, dst_v, sem = slot
        pltpu.make_async_copy(idxj_h.at[pl.ds(0, CB)], idxj_v, sem).wait()
        pltpu.make_async_copy(idxj_h.at[pl.ds(0, CB)], idxq_v, sem).wait()
        pltpu.make_async_copy(idxj_h.at[pl.ds(0, CB)], dst_v, sem).wait()

    def _fire(slot, rbuf):
        idxj_v, idxq_v, _, _ = slot
        qi_v, kj_v, rows, sem = rbuf
        pltpu.async_copy(qflat.at[idxq_v], qi_v, sem)
        pltpu.async_copy(kflat.at[idxj_v], kj_v, sem)
        pltpu.async_copy(xwf.at[idxj_v], rows, sem)

    def _drain(rbuf):
        qi_v, kj_v, rows, sem = rbuf
        pltpu.make_async_copy(qflat.at[pl.ds(0, CB)], qi_v, sem).wait()
        pltpu.make_async_copy(qflat.at[pl.ds(0, CB)], kj_v, sem).wait()
        pltpu.make_async_copy(xwf.at[pl.ds(0, CB)], rows, sem).wait()

    def _compute(rbuf, slot, sbuf):
        qi_v, kj_v, rows, _ = rbuf
        srows, ssem = sbuf
        dst_v = slot[2]
        for g in range(CB // 16):
            qi = qi_v[g * 16:(g + 1) * 16]
            kj = kj_v[g * 16:(g + 1) * 16]
            s = qi + kj
            a = jnp.where(s >= 0.0, s, 0.2 * s)
            ex = jnp.exp(a)
            d16 = dst_v[g * 16:(g + 1) * 16]
            for l in range(16):
                i = g * 16 + l
                sv = jnp.full((16,), ex[l], jnp.float32)
                for h in range(DIM // 16):
                    srows[i, h * 16:(h + 1) * 16] = (
                        rows[i, h * 16:(h + 1) * 16] * sv)
                d = d16[l]
                wb = (d >> 4) << 4
                v = den_v[pl.ds(wb, 16)]
                den_v[pl.ds(wb, 16)] = v + jnp.where(lane == d - wb, sv, 0.0)
        pltpu.async_copy(srows, accum.at[dst_v], ssem, add=True)

    slots = [(idxj_a, idxq_a, dst_a, sem_l0), (idxj_b, idxq_b, dst_b, sem_l1),
             (idxj_c, idxq_c, dst_c, sem_l2), (idxj_d, idxq_d, dst_d, sem_l3)]
    rbufs = [(qi_a, kj_a, rows_a, sem_a), (qi_b, kj_b, rows_b, sem_b)]
    sbufs = [(srows_a, sem_s0), (srows_b, sem_s1)]

    def _sdrain(sbuf):
        srows, ssem = sbuf
        pltpu.make_async_copy(srows, accum.at[prime_idx], ssem).wait()

    pltpu.async_copy(srows_a, accum.at[prime_idx], sem_s0, add=True)
    pltpu.async_copy(srows_b, accum.at[prime_idx], sem_s1, add=True)
    _lin(0, slots[0])
    _lin(1, slots[1])
    _lin_drain(slots[0])
    _fire(slots[0], rbufs[0])

    def _quad(u, c):
        for kk in range(4):
            t = 4 * u + kk
            _sdrain(sbufs[kk % 2])
            _lin(t + 2, slots[(kk + 2) % 4])
            _lin_drain(slots[(kk + 1) % 4])
            _fire(slots[(kk + 1) % 4], rbufs[(kk + 1) % 2])
            _drain(rbufs[kk % 2])
            _compute(rbufs[kk % 2], slots[kk % 4], sbufs[kk % 2])
        return c

    lax.fori_loop(0, CHUNKS // 4, _quad, 0)
    _drain(rbufs[0])
    _lin_drain(slots[1])
    _sdrain(sbufs[0])
    _sdrain(sbufs[1])

    pltpu.sync_copy(den_v, den_out.at[w])
    plsc.subcore_barrier()

    for b in range(RPS // CB):
        off = sid * RPS + b * CB
        pltpu.sync_copy(accum.at[pl.ds(off, CB)], srows_a)
        pltpu.sync_copy(srows_a, part.at[cid, pl.ds(off, CB)])


_sc_call = pl.kernel(
    _sc_body,
    out_type=[
        jax.ShapeDtypeStruct((NC, NPAD, DIM), jnp.float32),
        jax.ShapeDtypeStruct((NW, NPAD), jnp.float32),
    ],
    mesh=plsc.VectorSubcoreMesh(core_axis_name="c", subcore_axis_name="s"),
    scratch_types=(
        [pltpu.VMEM((CB,), jnp.int32)] * 13
        + [pltpu.VMEM((CB,), jnp.float32)] * 4
        + [pltpu.VMEM((CB, DIM), jnp.float32)] * 4
        + [pltpu.VMEM((NPAD,), jnp.float32),
           pltpu.VMEM_SHARED((NPAD, DIM), jnp.float32)]
        + [pltpu.SemaphoreType.DMA] * 8
    ),
)


def _fin_body(part_ref, den_ref, bias_ref, out_ref):
    num = part_ref[0] + part_ref[1]
    den = jnp.sum(den_ref[...], axis=1, keepdims=True)
    out_ref[...] = jnp.maximum(num / (den + 1e-16) + bias_ref[...], 0.0)


_fin_call = pl.pallas_call(
    _fin_body,
    grid=(NT,),
    in_specs=[
        pl.BlockSpec((NC, BN, DIM), lambda t: (0, t, 0)),
        pl.BlockSpec((BN, NW), lambda t: (t, 0)),
        pl.BlockSpec((1, DIM), lambda t: (0, 0)),
    ],
    out_specs=pl.BlockSpec((BN, DIM), lambda t: (t, 0)),
    out_shape=jax.ShapeDtypeStruct((NPAD, DIM), jnp.float32),
)


def kernel(adj, features, edge_type, weight, q, k, bias):
    src = adj[0]
    dst = adj[1]
    featp = jnp.zeros((NPAD, DIM), jnp.float32).at[:N].set(features)
    qk2 = jnp.concatenate([q, k], axis=1)
    xw, qnk = _xw_call(featp, weight, qk2)
    xwf = xw.reshape(R * NPAD, DIM)
    qflat = qnk[:, :, 0].reshape(R * NPAD)
    kflat = qnk[:, :, 1].reshape(R * NPAD)
    pad = EP + 2 * CB - E
    srcp = jnp.concatenate([src, jnp.zeros((pad,), jnp.int32)])
    dstp = jnp.concatenate([dst, jnp.full((pad,), NPAD - 1, jnp.int32)])
    relp = jnp.concatenate([edge_type, jnp.zeros((pad,), jnp.int32)])
    idxj = relp * NPAD + srcp
    idxq = relp * NPAD + dstp
    part, den_all = _sc_call(xwf, qflat, kflat, idxj, idxq, dstp)
    out = _fin_call(part, den_all.T, bias.reshape(1, DIM))
    return out[:N]
